# baseline (speedup 1.0000x reference)
import jax
import jax.numpy as jnp
from jax.experimental import pallas as pl

B, N_EX, N_CON, D, H, OUT = 8, 2048, 128, 128, 4, 256
DH = D // H
NCH = 8


def _enc_kernel(p_ref, exer_ref, lam_ref, concept_ref, q_ref, resp_ref,
                respT_ref, wqT_ref, bq_ref, wk_ref, bk_ref, wv_ref, bv_ref,
                mapw_ref, mapb_ref, out_ref):
    f32 = jnp.float32
    bf16 = jnp.bfloat16

    mqT = jnp.dot(wqT_ref[...], respT_ref[...],
                  preferred_element_type=f32) + bq_ref[...]
    mv = jnp.dot(resp_ref[...], wv_ref[...],
                 preferred_element_type=f32) + bv_ref[...]

    r_i = jax.lax.broadcasted_iota(jnp.int32, (D, NCH), 0)
    j_i = jax.lax.broadcasted_iota(jnp.int32, (D, NCH), 1)
    headok = (r_i // DH == j_i % H).astype(f32)
    Mq = jnp.where(j_i < H, mqT[:, 1:2], mqT[:, 2:3]) * headok

    scale = 1.0 / (DH ** 0.5)
    G = jnp.dot(wk_ref[...], Mq, preferred_element_type=f32) * scale
    CG = jnp.dot(concept_ref[...], G, preferred_element_type=f32)
    b_s = jnp.dot(bk_ref[...], Mq, preferred_element_type=f32) * scale

    Qm = q_ref[...]
    ones = jnp.ones((N_CON, NCH), bf16)
    sq = jnp.dot(Qm, CG.astype(bf16), preferred_element_type=f32)
    ccnt = jnp.dot(Qm, ones, preferred_element_type=f32)
    se = jnp.dot(exer_ref[...], G.astype(bf16), preferred_element_type=f32)
    S = se + lam_ref[...] * (sq / ccnt) + b_s

    w = jnp.exp(S - jnp.max(S, axis=0, keepdims=True)).astype(bf16)

    p = p_ref[...]
    ind1 = (p == 1).astype(bf16)
    ind2 = (p == 2).astype(bf16)
    ind_st = jnp.concatenate([ind1, ind2], axis=0)
    E = jnp.dot(ind_st, w, preferred_element_type=f32)
    e_top = E[0:B]
    e_bot = E[B:2 * B]
    den = e_top + e_bot
    sden = jnp.where(den > 0.0, den, 1.0)
    at = e_top / sden
    ab = e_bot / sden

    jj = jax.lax.broadcasted_iota(jnp.int32, (NCH, D), 0)
    rr = jax.lax.broadcasted_iota(jnp.int32, (NCH, D), 1)
    hh = (rr // DH == jj % H)
    selT1 = (hh & (jj < H)).astype(f32)
    selT2 = (hh & (jj >= H)).astype(f32)

    v1 = mv[1:2, :]
    v2 = mv[2:3, :]
    theta1 = (jnp.dot(at, selT1, preferred_element_type=f32) * v1
              + jnp.dot(ab, selT1, preferred_element_type=f32) * v2)
    theta2 = (jnp.dot(at, selT2, preferred_element_type=f32) * v1
              + jnp.dot(ab, selT2, preferred_element_type=f32) * v2)

    ns = jnp.sum(ind_st.astype(f32), axis=1, keepdims=True)
    n1 = ns[0:B]
    n2 = ns[B:2 * B]
    avg = (n1 * theta1 + n2 * theta2) / jnp.maximum(n1 + n2, 1.0)
    logits = jnp.dot(avg, mapw_ref[...], preferred_element_type=f32) + mapb_ref[...]
    out_ref[...] = jax.nn.sigmoid(logits)


def kernel(p_matrix, exer_emb, exer_lam, concept_emb, Q_matrix, resp_emb,
           Wq, bq, Wk, bk, Wv, bv, er_W, er_b, map_W, map_b):
    del er_W, er_b
    args = (p_matrix.astype(jnp.int32),
            exer_emb.astype(jnp.bfloat16), exer_lam, concept_emb,
            Q_matrix.astype(jnp.bfloat16), resp_emb,
            resp_emb.T, Wq.T, bq.reshape(D, 1),
            Wk, bk.reshape(1, D), Wv, bv.reshape(1, D),
            map_W, map_b.reshape(1, OUT))
    return pl.pallas_call(
        _enc_kernel,
        out_shape=jax.ShapeDtypeStruct((B, OUT), jnp.float32),
    )(*args)


# speedup vs baseline: 1.7763x; 1.7763x over previous
import jax
import jax.numpy as jnp
from jax.experimental import pallas as pl

B, N_EX, N_CON, D, H, OUT = 8, 2048, 128, 128, 4, 256
DH = D // H
NCH = 8


def _enc_kernel(p_ref, exer_ref, lam_ref, concept_ref, q_ref, resp_ref,
                wq_ref, bq_ref, wk_ref, bk_ref, wv_ref, bv_ref,
                mapw_ref, mapb_ref, out_ref):
    f32 = jnp.float32

    mqT = jax.lax.dot_general(
        wq_ref[...], resp_ref[...], (((0,), (1,)), ((), ())),
        preferred_element_type=f32) + bq_ref[...]
    mv = jnp.dot(resp_ref[...], wv_ref[...],
                 preferred_element_type=f32) + bv_ref[...]

    r_i = jax.lax.broadcasted_iota(jnp.int32, (D, NCH), 0)
    j_i = jax.lax.broadcasted_iota(jnp.int32, (D, NCH), 1)
    headok = (r_i // DH == j_i % H).astype(f32)
    Mq = jnp.where(j_i < H, mqT[:, 1:2], mqT[:, 2:3]) * headok

    scale = 1.0 / (DH ** 0.5)
    G = jnp.dot(wk_ref[...], Mq, preferred_element_type=f32) * scale
    CG = jnp.dot(concept_ref[...], G, preferred_element_type=f32)
    b_s = jnp.dot(bk_ref[...], Mq, preferred_element_type=f32) * scale

    Qm = q_ref[...]
    ones = jnp.ones((N_CON, NCH), f32)
    sq = jnp.dot(Qm, CG, preferred_element_type=f32)
    ccnt = jnp.dot(Qm, ones, preferred_element_type=f32)
    se = jnp.dot(exer_ref[...], G, preferred_element_type=f32)
    S = se + lam_ref[...] * (sq / ccnt) + b_s

    w = jnp.exp(S - jnp.max(S, axis=0, keepdims=True))

    p = p_ref[...]
    ind1 = (p == 1).astype(f32)
    ind2 = (p == 2).astype(f32)
    ind_st = jnp.concatenate([ind1, ind2], axis=0)
    E = jnp.dot(ind_st, w, preferred_element_type=f32)
    e_top = E[0:B]
    e_bot = E[B:2 * B]
    den = e_top + e_bot
    sden = jnp.where(den > 0.0, den, 1.0)
    at = e_top / sden
    ab = e_bot / sden

    jj = jax.lax.broadcasted_iota(jnp.int32, (NCH, D), 0)
    rr = jax.lax.broadcasted_iota(jnp.int32, (NCH, D), 1)
    hh = (rr // DH == jj % H)
    selT1 = (hh & (jj < H)).astype(f32)
    selT2 = (hh & (jj >= H)).astype(f32)

    v1 = mv[1:2, :]
    v2 = mv[2:3, :]
    theta1 = (jnp.dot(at, selT1, preferred_element_type=f32) * v1
              + jnp.dot(ab, selT1, preferred_element_type=f32) * v2)
    theta2 = (jnp.dot(at, selT2, preferred_element_type=f32) * v1
              + jnp.dot(ab, selT2, preferred_element_type=f32) * v2)

    ns = jnp.sum(ind_st, axis=1, keepdims=True)
    n1 = ns[0:B]
    n2 = ns[B:2 * B]
    avg = (n1 * theta1 + n2 * theta2) / jnp.maximum(n1 + n2, 1.0)
    logits = jnp.dot(avg, mapw_ref[...], preferred_element_type=f32) + mapb_ref[...]
    out_ref[...] = jax.nn.sigmoid(logits)


def kernel(p_matrix, exer_emb, exer_lam, concept_emb, Q_matrix, resp_emb,
           Wq, bq, Wk, bk, Wv, bv, er_W, er_b, map_W, map_b):
    del er_W, er_b
    args = (p_matrix.astype(jnp.int32), exer_emb, exer_lam, concept_emb,
            Q_matrix, resp_emb,
            Wq, bq.reshape(D, 1), Wk, bk.reshape(1, D), Wv, bv.reshape(1, D),
            map_W, map_b.reshape(1, OUT))
    return pl.pallas_call(
        _enc_kernel,
        out_shape=jax.ShapeDtypeStruct((B, OUT), jnp.float32),
    )(*args)
